# Initial kernel scaffold; baseline (speedup 1.0000x reference)
import jax
import jax.numpy as jnp
from jax.experimental import pallas as pl


def kernel(x, conn, W1, Wg):
    raise NotImplementedError("write your pallas kernel here")


# speedup vs baseline: 4.8570x; 4.8570x over previous
import functools

import jax
import jax.numpy as jnp
from jax import lax
from jax.experimental import pallas as pl
from jax.experimental.pallas import tpu as pltpu
from jax.experimental.pallas import tpu_sc as plsc

_NC = 2
_NS = 16
_NW = _NC * _NS
_CH = 128
_RB = 128


def _mm_body(x_ref, w1t_ref, wgt_ref, out_ref):
    h = jnp.maximum(
        jnp.dot(x_ref[...], w1t_ref[...], preferred_element_type=jnp.float32), 0.0)
    out_ref[...] = jnp.dot(h, wgt_ref[...], preferred_element_type=jnp.float32)


def _norm_body(acc_ref, out_ref):
    s = acc_ref[0] + acc_ref[1]
    nrm = jnp.sqrt(jnp.sum(s * s, axis=1, keepdims=True))
    out_ref[...] = s / nrm


def _make_sc_scatter(n_rows, n_pad, d, n_chunks):
    mesh = plsc.VectorSubcoreMesh(core_axis_name="c", subcore_axis_name="s")
    rows_per_tile = n_pad // _NS
    n_copies = rows_per_tile // _RB

    @functools.partial(
        pl.kernel,
        mesh=mesh,
        out_type=jax.ShapeDtypeStruct((_NC, n_pad, d), jnp.float32),
        scratch_types=[
            pltpu.VMEM((n_chunks, _CH), jnp.int32),
            pltpu.VMEM((n_chunks, _CH), jnp.int32),
            pltpu.VMEM((_CH, d), jnp.float32),
            pltpu.VMEM_SHARED((n_pad, d), jnp.float32),
            pltpu.SemaphoreType.DMA,
        ],
    )
    def sc_kernel(hw_hbm, src_hbm, dst_hbm, out_hbm, src_v, dst_v, rows_v, acc, sem):
        cid = lax.axis_index("c")
        sid = lax.axis_index("s")
        wid = cid * _NS + sid

        pltpu.sync_copy(src_hbm.at[wid], src_v)
        pltpu.sync_copy(dst_hbm.at[wid], dst_v)

        zvec = jnp.zeros((16,), jnp.float32)

        def zstore(i, _):
            r = i // (d // 16)
            l = (i % (d // 16)) * 16
            rows_v[r, pl.ds(l, 16)] = zvec
            return 0

        lax.fori_loop(0, _RB * (d // 16), zstore, 0)

        def zcopy(k, _):
            pltpu.sync_copy(rows_v, acc.at[pl.ds(sid * rows_per_tile + k * _RB, _RB)])
            return 0

        lax.fori_loop(0, n_copies, zcopy, 0)
        plsc.subcore_barrier()

        def body(j, _):
            pltpu.async_copy(hw_hbm.at[src_v.at[j]], rows_v, sem).wait()
            pltpu.sync_copy(rows_v, acc.at[dst_v.at[j]], add=True)
            return 0

        lax.fori_loop(0, n_chunks, body, 0)
        plsc.subcore_barrier()

        def wb(k, _):
            off = sid * rows_per_tile + k * _RB
            pltpu.sync_copy(acc.at[pl.ds(off, _RB)], rows_v)
            pltpu.sync_copy(rows_v, out_hbm.at[cid].at[pl.ds(off, _RB)])
            return 0

        lax.fori_loop(0, n_copies, wb, 0)

    return sc_kernel


def kernel(x, conn, W1, Wg):
    n, d = x.shape
    e = conn.shape[1]

    blk = 1000
    n_blk = n // blk
    hw = pl.pallas_call(
        _mm_body,
        grid=(n_blk,),
        in_specs=[
            pl.BlockSpec((blk, d), lambda i: (i, 0)),
            pl.BlockSpec((d, d), lambda i: (0, 0)),
            pl.BlockSpec((d, d), lambda i: (0, 0)),
        ],
        out_specs=pl.BlockSpec((blk, d), lambda i: (i, 0)),
        out_shape=jax.ShapeDtypeStruct((n, d), jnp.float32),
    )(x, W1.T, Wg.T)

    n_chunks = -(-e // (_NW * _CH))
    e_pad = _NW * n_chunks * _CH
    n_pad = -(-(n + 1) // (_NS * _RB)) * (_NS * _RB)
    src = jnp.concatenate([conn[0], jnp.zeros((e_pad - e,), jnp.int32)])
    dst = jnp.concatenate([conn[1], jnp.full((e_pad - e,), n, jnp.int32)])
    srcb = src.reshape(_NW, n_chunks, _CH)
    dstb = dst.reshape(_NW, n_chunks, _CH)
    partials = _make_sc_scatter(n, n_pad, d, n_chunks)(hw, srcb, dstb)

    out = pl.pallas_call(
        _norm_body,
        grid=(n_blk,),
        in_specs=[pl.BlockSpec((_NC, blk, d), lambda i: (0, i, 0))],
        out_specs=pl.BlockSpec((blk, d), lambda i: (i, 0)),
        out_shape=jax.ShapeDtypeStruct((n, d), jnp.float32),
    )(partials)
    return out
